# baseline (speedup 1.0000x reference)
import jax
import jax.numpy as jnp
from jax.experimental import pallas as pl
from jax.experimental.pallas import tpu as pltpu

_NE = 8
_K = 2
_BLK = 512
_NBUF = 12


def _top2_softmax(logits):
    iota = jax.lax.broadcasted_iota(jnp.int32, logits.shape, 1)
    m1 = jnp.max(logits, axis=-1, keepdims=True)
    i1 = jnp.min(jnp.where(logits == m1, iota, _NE), axis=-1, keepdims=True)
    masked = jnp.where(iota == i1, jnp.float32(-jnp.inf), logits)
    m2 = jnp.max(masked, axis=-1, keepdims=True)
    i2 = jnp.min(jnp.where(masked == m2, iota, _NE), axis=-1, keepdims=True)
    t = jnp.exp(m2 - m1)
    denom = 1.0 + t
    probs = jnp.concatenate([1.0 / denom, t / denom], axis=-1)
    idx = jnp.concatenate([i1, i2], axis=-1)
    return probs, idx


def _router_kernel(h_hbm, w_ref, probs_ref, idx_ref, buf, sems):
    n = probs_ref.shape[0]
    steps = n // _BLK
    w = w_ref[...]

    def start(i):
        pltpu.make_async_copy(
            h_hbm.at[pl.ds(i * _BLK, _BLK), :], buf.at[i % _NBUF], sems.at[i % _NBUF]
        ).start()

    for i in range(_NBUF - 1):
        start(i)
    for i in range(steps):
        pltpu.make_async_copy(
            h_hbm.at[pl.ds(i * _BLK, _BLK), :], buf.at[i % _NBUF], sems.at[i % _NBUF]
        ).wait()
        h = buf[i % _NBUF]
        logits = jax.lax.dot_general(
            h, w, (((1,), (1,)), ((), ())), preferred_element_type=jnp.float32
        )
        probs, idx = _top2_softmax(logits)
        probs_ref[pl.ds(i * _BLK, _BLK), :] = probs
        idx_ref[pl.ds(i * _BLK, _BLK), :] = idx
        if i + _NBUF - 1 < steps:
            start(i + _NBUF - 1)


@jax.jit
def kernel(hidden_states, weight):
    h = hidden_states.reshape(-1, hidden_states.shape[-1])
    n, hd = h.shape
    probs, idx = pl.pallas_call(
        _router_kernel,
        in_specs=[
            pl.BlockSpec(memory_space=pltpu.MemorySpace.HBM),
            pl.BlockSpec(memory_space=pltpu.MemorySpace.VMEM),
        ],
        out_specs=[
            pl.BlockSpec(memory_space=pltpu.MemorySpace.VMEM),
            pl.BlockSpec(memory_space=pltpu.MemorySpace.VMEM),
        ],
        out_shape=[
            jax.ShapeDtypeStruct((n, _K), jnp.float32),
            jax.ShapeDtypeStruct((n, _K), jnp.int32),
        ],
        scratch_shapes=[
            pltpu.VMEM((_NBUF, _BLK, hd), jnp.float32),
            pltpu.SemaphoreType.DMA((_NBUF,)),
        ],
        compiler_params=pltpu.CompilerParams(
            vmem_limit_bytes=100 * 1024 * 1024,
        ),
    )(h, weight)
    return probs, idx


# speedup vs baseline: 1.1680x; 1.1680x over previous
import jax
import jax.numpy as jnp
from jax.experimental import pallas as pl
from jax.experimental.pallas import tpu as pltpu

_NE = 8
_K = 2


def _top2_softmax(logits):
    iota = jax.lax.broadcasted_iota(jnp.int32, logits.shape, 1)
    m1 = jnp.max(logits, axis=-1, keepdims=True)
    i1 = jnp.min(jnp.where(logits == m1, iota, _NE), axis=-1, keepdims=True)
    masked = jnp.where(iota == i1, jnp.float32(-jnp.inf), logits)
    m2 = jnp.max(masked, axis=-1, keepdims=True)
    i2 = jnp.min(jnp.where(masked == m2, iota, _NE), axis=-1, keepdims=True)
    t = jnp.exp(m2 - m1)
    denom = 1.0 + t
    probs = jnp.concatenate([1.0 / denom, t / denom], axis=-1)
    idx = jnp.concatenate([i1, i2], axis=-1)
    return probs, idx


def _router_kernel(h_ref, w_ref, probs_ref, idx_ref):
    h = h_ref[...]
    w = w_ref[...]
    logits = jax.lax.dot_general(
        h, w, (((1,), (1,)), ((), ())), preferred_element_type=jnp.float32
    )
    probs, idx = _top2_softmax(logits)
    probs_ref[...] = probs
    idx_ref[...] = idx


@jax.jit
def kernel(hidden_states, weight):
    h = hidden_states.reshape(-1, hidden_states.shape[-1])
    n, hd = h.shape
    blk = 4096
    probs, idx = pl.pallas_call(
        _router_kernel,
        grid=(n // blk,),
        in_specs=[
            pl.BlockSpec((blk, hd), lambda i: (i, 0)),
            pl.BlockSpec((_NE, hd), lambda i: (0, 0)),
        ],
        out_specs=[
            pl.BlockSpec((blk, _K), lambda i: (i, 0)),
            pl.BlockSpec((blk, _K), lambda i: (i, 0)),
        ],
        out_shape=[
            jax.ShapeDtypeStruct((n, _K), jnp.float32),
            jax.ShapeDtypeStruct((n, _K), jnp.int32),
        ],
        compiler_params=pltpu.CompilerParams(
            dimension_semantics=("arbitrary",),
            vmem_limit_bytes=100 * 1024 * 1024,
        ),
    )(h, weight)
    return probs, idx
